# baseline (speedup 1.0000x reference)
import functools

import jax
import jax.numpy as jnp
from jax.experimental import pallas as pl
from jax.experimental.pallas import tpu as pltpu


def _gcn_stripe_kernel(x_ref, adj0_ref, adj1_ref, wt_ref, b_ref, out_ref,
                       h_ref):
    @pl.when(pl.program_id(0) == 0)
    def _project():
        h = jnp.dot(x_ref[...], wt_ref[...], preferred_element_type=jnp.float32)
        h_ref[...] = h + b_ref[...]

    bm = adj0_ref.shape[0]
    out_ref[:bm, :] = jnp.dot(adj0_ref[...], h_ref[...],
                              preferred_element_type=jnp.float32)
    out_ref[bm:, :] = jnp.dot(adj1_ref[...], h_ref[...],
                              preferred_element_type=jnp.float32)


@functools.partial(jax.jit, static_argnames=("bm",))
def _gcn(x, adj, wt, b, bm):
    n, d = x.shape
    grid = (n // (2 * bm),)
    return pl.pallas_call(
        _gcn_stripe_kernel,
        grid=grid,
        in_specs=[
            pl.BlockSpec((n, d), lambda i: (0, 0)),
            pl.BlockSpec((bm, n), lambda i: (2 * i, 0)),
            pl.BlockSpec((bm, n), lambda i: (2 * i + 1, 0)),
            pl.BlockSpec((d, d), lambda i: (0, 0)),
            pl.BlockSpec((1, d), lambda i: (0, 0)),
        ],
        out_specs=pl.BlockSpec((2 * bm, d), lambda i: (i, 0)),
        out_shape=jax.ShapeDtypeStruct((n, d), jnp.float32),
        scratch_shapes=[pltpu.VMEM((n, d), jnp.float32)],
        compiler_params=pltpu.CompilerParams(
            dimension_semantics=("arbitrary",),
        ),
    )(x, adj, adj, wt, b)


def _gcn_single_kernel(x_ref, adj_ref, wt_ref, b_ref, out_ref):
    h = jnp.dot(x_ref[...], wt_ref[...], preferred_element_type=jnp.float32)
    out_ref[...] = jnp.dot(adj_ref[...], h + b_ref[...],
                           preferred_element_type=jnp.float32)


@jax.jit
def _gcn_fallback(x, adj, wt, b):
    n, d = x.shape
    return pl.pallas_call(
        _gcn_single_kernel,
        out_shape=jax.ShapeDtypeStruct((n, d), jnp.float32),
    )(x, adj, wt, b)


def kernel(x, adj, W, b, is_sparse):
    n, d = x.shape
    wt = W.T
    b2 = b.reshape(1, d)
    if n % 400 == 0:
        return _gcn(x, adj, wt, b2, 200)
    return _gcn_fallback(x, adj, wt, b2)


# speedup vs baseline: 1.0234x; 1.0234x over previous
import functools

import jax
import jax.numpy as jnp
from jax.experimental import pallas as pl
from jax.experimental.pallas import tpu as pltpu


def _gcn_stripe_kernel(x_ref, adj_ref, wt_ref, b_ref, out_ref, h_ref):
    @pl.when(pl.program_id(0) == 0)
    def _project():
        h = jnp.dot(x_ref[...], wt_ref[...], preferred_element_type=jnp.float32)
        h_ref[...] = h + b_ref[...]

    out_ref[...] = jnp.dot(adj_ref[...], h_ref[...],
                           preferred_element_type=jnp.float32)


@functools.partial(jax.jit, static_argnames=("bm",))
def _gcn(x, adj, wt, b, bm):
    n, d = x.shape
    grid = (n // bm,)
    return pl.pallas_call(
        _gcn_stripe_kernel,
        grid=grid,
        in_specs=[
            pl.BlockSpec((n, d), lambda i: (0, 0)),
            pl.BlockSpec((bm, n), lambda i: (i, 0)),
            pl.BlockSpec((d, d), lambda i: (0, 0)),
            pl.BlockSpec((1, d), lambda i: (0, 0)),
        ],
        out_specs=pl.BlockSpec((bm, d), lambda i: (i, 0)),
        out_shape=jax.ShapeDtypeStruct((n, d), jnp.float32),
        scratch_shapes=[pltpu.VMEM((n, d), jnp.float32)],
        compiler_params=pltpu.CompilerParams(
            dimension_semantics=("arbitrary",),
        ),
    )(x, adj, wt, b)


def kernel(x, adj, W, b, is_sparse):
    n, d = x.shape
    bm = 400 if n % 400 == 0 else n
    wt = W.T
    b2 = b.reshape(1, d)
    return _gcn(x, adj, wt, b2, bm)


# speedup vs baseline: 1.0350x; 1.0114x over previous
import functools

import jax
import jax.numpy as jnp
from jax.experimental import pallas as pl
from jax.experimental.pallas import tpu as pltpu


def _gcn_stripe_kernel(x_ref, adj_ref, w_ref, b_ref, out_ref, h_ref):
    @pl.when(pl.program_id(0) == 0)
    def _project():
        h = jax.lax.dot_general(
            x_ref[...], w_ref[...], (((1,), (1,)), ((), ())),
            preferred_element_type=jnp.float32)
        h_ref[...] = h + b_ref[...]

    out_ref[...] = jnp.dot(adj_ref[...], h_ref[...],
                           preferred_element_type=jnp.float32)


@functools.partial(jax.jit, static_argnames=("bm",))
def _gcn(x, adj, w, b, bm):
    n, d = x.shape
    grid = (n // bm,)
    return pl.pallas_call(
        _gcn_stripe_kernel,
        grid=grid,
        in_specs=[
            pl.BlockSpec((n, d), lambda i: (0, 0)),
            pl.BlockSpec((bm, n), lambda i: (i, 0)),
            pl.BlockSpec((d, d), lambda i: (0, 0)),
            pl.BlockSpec((1, d), lambda i: (0, 0)),
        ],
        out_specs=pl.BlockSpec((bm, d), lambda i: (i, 0)),
        out_shape=jax.ShapeDtypeStruct((n, d), jnp.float32),
        scratch_shapes=[pltpu.VMEM((n, d), jnp.float32)],
        compiler_params=pltpu.CompilerParams(
            dimension_semantics=("arbitrary",),
        ),
    )(x, adj, w, b)


def kernel(x, adj, W, b, is_sparse):
    n, d = x.shape
    bm = 400 if n % 400 == 0 else n
    b2 = b.reshape(1, d)
    return _gcn(x, adj, W, b2, bm)


# speedup vs baseline: 1.0351x; 1.0001x over previous
import functools

import jax
import jax.numpy as jnp
from jax.experimental import pallas as pl
from jax.experimental.pallas import tpu as pltpu


def _gcn_stripe_kernel(x_ref, adj_ref, w_ref, b_ref, out_ref, h_ref):
    @pl.when(pl.program_id(0) == 0)
    def _project():
        h = jax.lax.dot_general(
            x_ref[...], w_ref[...], (((1,), (1,)), ((), ())),
            preferred_element_type=jnp.float32)
        h_ref[...] = h + b_ref[...][None, :]

    out_ref[...] = jnp.dot(adj_ref[...], h_ref[...],
                           preferred_element_type=jnp.float32)


@functools.partial(jax.jit, static_argnames=("bm",))
def _gcn(x, adj, w, b, bm):
    n, d = x.shape
    grid = (n // bm,)
    return pl.pallas_call(
        _gcn_stripe_kernel,
        grid=grid,
        in_specs=[
            pl.BlockSpec((n, d), lambda i: (0, 0)),
            pl.BlockSpec((bm, n), lambda i: (i, 0)),
            pl.BlockSpec((d, d), lambda i: (0, 0)),
            pl.BlockSpec((d,), lambda i: (0,)),
        ],
        out_specs=pl.BlockSpec((bm, d), lambda i: (i, 0)),
        out_shape=jax.ShapeDtypeStruct((n, d), jnp.float32),
        scratch_shapes=[pltpu.VMEM((n, d), jnp.float32)],
        compiler_params=pltpu.CompilerParams(
            dimension_semantics=("arbitrary",),
        ),
    )(x, adj, w, b)


def kernel(x, adj, W, b, is_sparse):
    n, d = x.shape
    bm = 400 if n % 400 == 0 else n
    return _gcn(x, adj, W, b, bm)
